# Initial kernel scaffold; baseline (speedup 1.0000x reference)
import jax
import jax.numpy as jnp
from jax.experimental import pallas as pl


def kernel(x, W):
    raise NotImplementedError("write your pallas kernel here")


# speedup vs baseline: 9.8138x; 9.8138x over previous
import functools

import jax
import jax.numpy as jnp
from jax.experimental import pallas as pl
from jax.experimental.pallas import tpu as pltpu

_K = 8
_CAPACITY_FACTOR = 1.25


def _phase_a_body(x_ref, w_ref, rw_ref, tkw_ref, tki_ref, hist_ref, zsum_ref,
                  *, n_experts):
    i = pl.program_id(0)

    @pl.when(i == 0)
    def _init():
        hist_ref[...] = jnp.zeros_like(hist_ref)
        zsum_ref[...] = jnp.zeros_like(zsum_ref)

    logits = jax.lax.dot_general(
        x_ref[...], w_ref[...], (((1,), (1,)), ((), ())),
        preferred_element_type=jnp.float32)
    zsum_ref[...] += jnp.sum(logits * logits)

    m = jnp.max(logits, axis=1, keepdims=True)
    ex = jnp.exp(logits - m)
    p = ex / jnp.sum(ex, axis=1, keepdims=True)
    rw_ref[...] = p

    iota = jax.lax.broadcasted_iota(jnp.int32, p.shape, 1)
    work = p
    wcols, icols, hrows = [], [], []
    for _ in range(_K):
        mk = jnp.max(work, axis=1, keepdims=True)
        idx = jnp.min(jnp.where(work == mk, iota, n_experts), axis=1,
                      keepdims=True)
        oh = iota == idx
        wcols.append(mk)
        icols.append(idx)
        hrows.append(jnp.sum(oh.astype(jnp.float32), axis=0))
        work = jnp.where(oh, -1.0, work)
    wmat = jnp.concatenate(wcols, axis=1)
    tki_ref[...] = jnp.concatenate(icols, axis=1)
    tkw_ref[...] = wmat / (jnp.sum(wmat, axis=1, keepdims=True) + 1e-8)
    hist_ref[...] += jnp.stack(hrows, axis=0)


def _phase_b_body(tkw_ref, tki_ref, hist_ref, zsum_ref, dm_ref, loss_ref,
                  colsum_ref, *, n_experts, n_tokens, capacity):
    i = pl.program_id(0)
    e = n_experts

    hist = hist_ref[...]
    counts = jnp.zeros((1, e), jnp.float32)
    allowed_rows = []
    for k in range(_K):
        a = (counts < capacity).astype(jnp.float32)
        allowed_rows.append(a)
        counts = counts + hist[k:k + 1, :] * a

    iota_e = jax.lax.broadcasted_iota(jnp.int32, (1, e), 1)
    minc = jnp.min(counts, axis=1, keepdims=True)
    least = jnp.min(jnp.where(counts == minc, iota_e, e), axis=1,
                    keepdims=True)

    tki = tki_ref[...]
    tkw = tkw_ref[...]
    iota64 = jax.lax.broadcasted_iota(jnp.int32, tkw.shape[:1] + (e,), 1)
    dm = jnp.zeros(iota64.shape, jnp.float32)
    for k in range(_K):
        oh = (iota64 == tki[:, k:k + 1]).astype(jnp.float32)
        dm = dm + oh * allowed_rows[k] * tkw[:, k:k + 1]
    rs = jnp.sum(dm, axis=1, keepdims=True)
    dm = jnp.where((rs == 0.0) & (iota64 == least), 1.0, dm)
    dm = dm / (jnp.sum(dm, axis=1, keepdims=True) + 1e-8)
    dm_ref[...] = dm

    @pl.when(i == 0)
    def _init():
        colsum_ref[...] = jnp.zeros_like(colsum_ref)

    colsum_ref[...] += jnp.sum(dm, axis=0, keepdims=True)

    @pl.when(i == pl.num_programs(0) - 1)
    def _final():
        counts2 = colsum_ref[...]
        target = n_tokens * _K / e
        lb = jnp.mean(jnp.square(counts2 / n_tokens - target / n_tokens))
        z = zsum_ref[0, 0] / (n_tokens * e)
        loss_ref[...] = jnp.full((1, 1), 0.0, jnp.float32) + (
            0.001 * z + 0.001 * lb)


@jax.jit
def kernel(x, W):
    b, s, d = x.shape
    n = b * s
    e = W.shape[0]
    capacity = int(_CAPACITY_FACTOR * n * _K / e)
    xf = x.reshape(n, d)

    bn_a = 512
    grid_a = n // bn_a
    rw, tkw, tki, hist, zsum = pl.pallas_call(
        functools.partial(_phase_a_body, n_experts=e),
        grid=(grid_a,),
        in_specs=[
            pl.BlockSpec((bn_a, d), lambda i: (i, 0)),
            pl.BlockSpec((e, d), lambda i: (0, 0)),
        ],
        out_specs=[
            pl.BlockSpec((bn_a, e), lambda i: (i, 0)),
            pl.BlockSpec((bn_a, _K), lambda i: (i, 0)),
            pl.BlockSpec((bn_a, _K), lambda i: (i, 0)),
            pl.BlockSpec((_K, e), lambda i: (0, 0)),
            pl.BlockSpec((1, 1), lambda i: (0, 0)),
        ],
        out_shape=[
            jax.ShapeDtypeStruct((n, e), jnp.float32),
            jax.ShapeDtypeStruct((n, _K), jnp.float32),
            jax.ShapeDtypeStruct((n, _K), jnp.int32),
            jax.ShapeDtypeStruct((_K, e), jnp.float32),
            jax.ShapeDtypeStruct((1, 1), jnp.float32),
        ],
    )(xf, W)

    bn_b = 1024
    grid_b = n // bn_b
    dm, loss = pl.pallas_call(
        functools.partial(_phase_b_body, n_experts=e, n_tokens=n,
                          capacity=capacity),
        grid=(grid_b,),
        in_specs=[
            pl.BlockSpec((bn_b, _K), lambda i: (i, 0)),
            pl.BlockSpec((bn_b, _K), lambda i: (i, 0)),
            pl.BlockSpec((_K, e), lambda i: (0, 0)),
            pl.BlockSpec((1, 1), lambda i: (0, 0)),
        ],
        out_specs=[
            pl.BlockSpec((bn_b, e), lambda i: (i, 0)),
            pl.BlockSpec((1, 1), lambda i: (0, 0)),
        ],
        out_shape=[
            jax.ShapeDtypeStruct((n, e), jnp.float32),
            jax.ShapeDtypeStruct((1, 1), jnp.float32),
        ],
        scratch_shapes=[pltpu.VMEM((1, e), jnp.float32)],
    )(tkw, tki, hist, zsum)

    return rw, dm, loss[0, 0]


# speedup vs baseline: 11.6385x; 1.1859x over previous
import functools

import jax
import jax.numpy as jnp
from jax.experimental import pallas as pl
from jax.experimental.pallas import tpu as pltpu

_K = 8
_CAPACITY_FACTOR = 1.25


def _phase_a_body(x_ref, w_ref, rw_ref, kmat_ref, wsum_ref, hist_ref,
                  zsum_ref, *, n_experts):
    i = pl.program_id(0)

    @pl.when(i == 0)
    def _init():
        hist_ref[...] = jnp.zeros_like(hist_ref)
        zsum_ref[...] = jnp.zeros_like(zsum_ref)

    logits = jax.lax.dot_general(
        x_ref[...], w_ref[...], (((1,), (1,)), ((), ())),
        preferred_element_type=jnp.float32)
    zsum_ref[...] += jnp.sum(logits * logits)

    m = jnp.max(logits, axis=1, keepdims=True)
    ex = jnp.exp(logits - m)
    p = ex / jnp.sum(ex, axis=1, keepdims=True)
    rw_ref[...] = p

    iota = jax.lax.broadcasted_iota(jnp.int32, p.shape, 1)
    work = p
    kacc = jnp.full(p.shape, float(_K), jnp.float32)
    wsum = jnp.zeros((p.shape[0], 1), jnp.float32)
    hrows = []
    for k in range(_K):
        mk = jnp.max(work, axis=1, keepdims=True)
        idx = jnp.min(jnp.where(work == mk, iota, n_experts), axis=1,
                      keepdims=True)
        oh = (iota == idx).astype(jnp.float32)
        wsum = wsum + mk
        kacc = kacc - (float(_K) - k) * oh
        hrows.append(jnp.sum(oh, axis=0))
        work = work - oh * (work + 1.0)
    rows0 = jnp.stack(hrows, axis=0)
    kmat_ref[...] = kacc
    wsum_ref[...] = wsum
    hist_ref[...] += rows0


def _phase_b_body(rw_ref, kmat_ref, wsum_ref, hist_ref, zsum_ref, dm_ref,
                  loss_ref, colsum_ref, *, n_experts, n_tokens, capacity):
    i = pl.program_id(0)
    e = n_experts

    hist = hist_ref[...]
    counts = jnp.zeros((1, e), jnp.float32)
    kmax = jnp.zeros((1, e), jnp.float32)
    for k in range(_K):
        a = (counts < capacity).astype(jnp.float32)
        kmax = kmax + a
        counts = counts + hist[k:k + 1, :] * a

    iota_e = jax.lax.broadcasted_iota(jnp.int32, (1, e), 1)
    minc = jnp.min(counts, axis=1, keepdims=True)
    least = jnp.min(jnp.where(counts == minc, iota_e, e), axis=1,
                    keepdims=True)

    p = rw_ref[...]
    kmat = kmat_ref[...]
    wsum = wsum_ref[...]
    allowed = (kmat < kmax).astype(jnp.float32)
    dm = p * allowed / (wsum + 1e-8)
    rs = jnp.sum(dm, axis=1, keepdims=True)
    iota64 = jax.lax.broadcasted_iota(jnp.int32, dm.shape, 1)
    dm = jnp.where((rs == 0.0) & (iota64 == least), 1.0, dm)
    dm = dm / (jnp.sum(dm, axis=1, keepdims=True) + 1e-8)
    dm_ref[...] = dm

    @pl.when(i == 0)
    def _init():
        colsum_ref[...] = jnp.zeros_like(colsum_ref)

    colsum_ref[...] += jnp.sum(dm, axis=0, keepdims=True)

    @pl.when(i == pl.num_programs(0) - 1)
    def _final():
        counts2 = colsum_ref[...]
        target = n_tokens * _K / e
        lb = jnp.mean(jnp.square(counts2 / n_tokens - target / n_tokens))
        z = zsum_ref[0, 0] / (n_tokens * e)
        loss_ref[...] = jnp.full((1, 1), 0.0, jnp.float32) + (
            0.001 * z + 0.001 * lb)


@jax.jit
def kernel(x, W):
    b, s, d = x.shape
    n = b * s
    e = W.shape[0]
    capacity = int(_CAPACITY_FACTOR * n * _K / e)
    xf = x.reshape(n, d)

    bn_a = 512
    grid_a = n // bn_a
    rw, kmat, wsum, hist, zsum = pl.pallas_call(
        functools.partial(_phase_a_body, n_experts=e),
        grid=(grid_a,),
        in_specs=[
            pl.BlockSpec((bn_a, d), lambda i: (i, 0)),
            pl.BlockSpec((e, d), lambda i: (0, 0)),
        ],
        out_specs=[
            pl.BlockSpec((bn_a, e), lambda i: (i, 0)),
            pl.BlockSpec((bn_a, e), lambda i: (i, 0)),
            pl.BlockSpec((bn_a, 1), lambda i: (i, 0)),
            pl.BlockSpec((_K, e), lambda i: (0, 0)),
            pl.BlockSpec((1, 1), lambda i: (0, 0)),
        ],
        out_shape=[
            jax.ShapeDtypeStruct((n, e), jnp.float32),
            jax.ShapeDtypeStruct((n, e), jnp.float32),
            jax.ShapeDtypeStruct((n, 1), jnp.float32),
            jax.ShapeDtypeStruct((_K, e), jnp.float32),
            jax.ShapeDtypeStruct((1, 1), jnp.float32),
        ],
    )(xf, W)

    bn_b = 1024
    grid_b = n // bn_b
    dm, loss = pl.pallas_call(
        functools.partial(_phase_b_body, n_experts=e, n_tokens=n,
                          capacity=capacity),
        grid=(grid_b,),
        in_specs=[
            pl.BlockSpec((bn_b, e), lambda i: (i, 0)),
            pl.BlockSpec((bn_b, e), lambda i: (i, 0)),
            pl.BlockSpec((bn_b, 1), lambda i: (i, 0)),
            pl.BlockSpec((_K, e), lambda i: (0, 0)),
            pl.BlockSpec((1, 1), lambda i: (0, 0)),
        ],
        out_specs=[
            pl.BlockSpec((bn_b, e), lambda i: (i, 0)),
            pl.BlockSpec((1, 1), lambda i: (0, 0)),
        ],
        out_shape=[
            jax.ShapeDtypeStruct((n, e), jnp.float32),
            jax.ShapeDtypeStruct((1, 1), jnp.float32),
        ],
        scratch_shapes=[pltpu.VMEM((1, e), jnp.float32)],
    )(rw, kmat, wsum, hist, zsum)

    return rw, dm, loss[0, 0]


# speedup vs baseline: 13.4989x; 1.1599x over previous
import functools

import jax
import jax.numpy as jnp
from jax.experimental import pallas as pl
from jax.experimental.pallas import tpu as pltpu

_K = 8
_CAPACITY_FACTOR = 1.25


def _phase_a_body(x_ref, w_ref, rw_ref, d0_ref, kmat_ref, hist_ref,
                  zsum_ref, *, n_experts):
    i = pl.program_id(0)

    @pl.when(i == 0)
    def _init():
        hist_ref[...] = jnp.zeros_like(hist_ref)
        zsum_ref[...] = jnp.zeros_like(zsum_ref)

    logits = jax.lax.dot_general(
        w_ref[...], x_ref[...], (((1,), (1,)), ((), ())),
        preferred_element_type=jnp.float32)
    zsum_ref[...] += jnp.sum(logits * logits)

    m = jnp.max(logits, axis=0, keepdims=True)
    ex = jnp.exp(logits - m)
    p = ex / jnp.sum(ex, axis=0, keepdims=True)
    rw_ref[...] = jnp.swapaxes(p, 0, 1)

    iota = jax.lax.broadcasted_iota(jnp.int32, p.shape, 0)
    work = p
    kacc = jnp.full(p.shape, float(_K), jnp.float32)
    wsum = jnp.zeros((1, p.shape[1]), jnp.float32)
    hcols = []
    for k in range(_K):
        mk = jnp.max(work, axis=0, keepdims=True)
        idx = jnp.min(jnp.where(work == mk, iota, n_experts), axis=0,
                      keepdims=True)
        oh = (iota == idx).astype(jnp.float32)
        wsum = wsum + mk
        kacc = kacc - (float(_K) - k) * oh
        hcols.append(jnp.sum(oh, axis=1, keepdims=True))
        work = work - oh * (work + 1.0)
    kmat_ref[...] = kacc
    d0_ref[...] = p / (wsum + 1e-8)
    hist_ref[...] += jnp.concatenate(hcols, axis=1)


def _phase_b_body(d0_ref, kmat_ref, hist_ref, zsum_ref, dm_ref, loss_ref,
                  colsum_ref, *, n_experts, n_tokens, capacity):
    i = pl.program_id(0)
    e = n_experts

    hist = hist_ref[...]
    counts = jnp.zeros((e, 1), jnp.float32)
    kmax = jnp.zeros((e, 1), jnp.float32)
    for k in range(_K):
        a = (counts < capacity).astype(jnp.float32)
        kmax = kmax + a
        counts = counts + hist[:, k:k + 1] * a

    iota_e = jax.lax.broadcasted_iota(jnp.int32, (e, 1), 0)
    minc = jnp.min(counts, axis=0, keepdims=True)
    least = jnp.min(jnp.where(counts == minc, iota_e, e), axis=0,
                    keepdims=True)

    d0 = d0_ref[...]
    kmat = kmat_ref[...]
    dm = d0 * (kmat < kmax).astype(jnp.float32)
    rs = jnp.sum(dm, axis=0, keepdims=True)
    iota_s = jax.lax.broadcasted_iota(jnp.int32, dm.shape, 0)
    dm = jnp.where((rs == 0.0) & (iota_s == least), 1.0, dm)
    dm = dm / (jnp.sum(dm, axis=0, keepdims=True) + 1e-8)
    dm_ref[...] = jnp.swapaxes(dm, 0, 1)

    @pl.when(i == 0)
    def _init():
        colsum_ref[...] = jnp.zeros_like(colsum_ref)

    colsum_ref[...] += jnp.sum(dm, axis=1, keepdims=True)

    @pl.when(i == pl.num_programs(0) - 1)
    def _final():
        counts2 = colsum_ref[...]
        target = n_tokens * _K / e
        lb = jnp.mean(jnp.square(counts2 / n_tokens - target / n_tokens))
        z = zsum_ref[0, 0] / (n_tokens * e)
        loss_ref[...] = jnp.full((1, 1), 0.0, jnp.float32) + (
            0.001 * z + 0.001 * lb)


@jax.jit
def kernel(x, W):
    b, s, d = x.shape
    n = b * s
    e = W.shape[0]
    capacity = int(_CAPACITY_FACTOR * n * _K / e)
    xf = x.reshape(n, d)

    bn_a = 512
    grid_a = n // bn_a
    rw, d0, kmat, hist, zsum = pl.pallas_call(
        functools.partial(_phase_a_body, n_experts=e),
        grid=(grid_a,),
        in_specs=[
            pl.BlockSpec((bn_a, d), lambda i: (i, 0)),
            pl.BlockSpec((e, d), lambda i: (0, 0)),
        ],
        out_specs=[
            pl.BlockSpec((bn_a, e), lambda i: (i, 0)),
            pl.BlockSpec((e, bn_a), lambda i: (0, i)),
            pl.BlockSpec((e, bn_a), lambda i: (0, i)),
            pl.BlockSpec((e, _K), lambda i: (0, 0)),
            pl.BlockSpec((1, 1), lambda i: (0, 0)),
        ],
        out_shape=[
            jax.ShapeDtypeStruct((n, e), jnp.float32),
            jax.ShapeDtypeStruct((e, n), jnp.float32),
            jax.ShapeDtypeStruct((e, n), jnp.float32),
            jax.ShapeDtypeStruct((e, _K), jnp.float32),
            jax.ShapeDtypeStruct((1, 1), jnp.float32),
        ],
        input_output_aliases={},
    )(xf, W)

    bn_b = 1024
    grid_b = n // bn_b
    dm, loss = pl.pallas_call(
        functools.partial(_phase_b_body, n_experts=e, n_tokens=n,
                          capacity=capacity),
        grid=(grid_b,),
        in_specs=[
            pl.BlockSpec((e, bn_b), lambda i: (0, i)),
            pl.BlockSpec((e, bn_b), lambda i: (0, i)),
            pl.BlockSpec((e, _K), lambda i: (0, 0)),
            pl.BlockSpec((1, 1), lambda i: (0, 0)),
        ],
        out_specs=[
            pl.BlockSpec((bn_b, e), lambda i: (i, 0)),
            pl.BlockSpec((1, 1), lambda i: (0, 0)),
        ],
        out_shape=[
            jax.ShapeDtypeStruct((n, e), jnp.float32),
            jax.ShapeDtypeStruct((1, 1), jnp.float32),
        ],
        scratch_shapes=[pltpu.VMEM((e, 1), jnp.float32)],
    )(d0, kmat, hist, zsum)

    return rw, dm, loss[0, 0]


# speedup vs baseline: 15.0010x; 1.1113x over previous
import functools

import jax
import jax.numpy as jnp
from jax.experimental import pallas as pl
from jax.experimental.pallas import tpu as pltpu

_K = 8
_CAPACITY_FACTOR = 1.25


def _phase_a_body(x_ref, w_ref, rw_ref, d0_ref, kmat_ref, hist_ref,
                  zsum_ref, *, n_experts):
    i = pl.program_id(0)

    @pl.when(i == 0)
    def _init():
        hist_ref[...] = jnp.zeros_like(hist_ref)
        zsum_ref[...] = jnp.zeros_like(zsum_ref)

    logits = jax.lax.dot_general(
        w_ref[...], x_ref[...], (((1,), (1,)), ((), ())),
        preferred_element_type=jnp.float32)
    zsum_ref[...] += jnp.sum(logits * logits)

    m = jnp.max(logits, axis=0, keepdims=True)
    ex = jnp.exp(logits - m)
    p = ex / jnp.sum(ex, axis=0, keepdims=True)
    rw_ref[...] = jnp.swapaxes(p, 0, 1)

    iota = jax.lax.broadcasted_iota(jnp.int32, p.shape, 0)
    work = p
    kacc = jnp.full(p.shape, float(_K), jnp.float32)
    wsum = jnp.zeros((1, p.shape[1]), jnp.float32)
    hcols = []
    for k in range(_K):
        mk = jnp.max(work, axis=0, keepdims=True)
        idx = jnp.min(jnp.where(work == mk, iota, n_experts), axis=0,
                      keepdims=True)
        oh = (iota == idx).astype(jnp.float32)
        wsum = wsum + mk
        kacc = kacc - (float(_K) - k) * oh
        hcols.append(jnp.sum(oh, axis=1, keepdims=True))
        work = work - oh * (work + 1.0)
    kmat_ref[...] = kacc
    d0_ref[...] = p / (wsum + 1e-8)
    hist_ref[...] += jnp.concatenate(hcols, axis=1)


def _phase_b_body(d0_ref, kmat_ref, hist_ref, zsum_ref, dm_ref, loss_ref,
                  colsum_ref, *, n_experts, n_tokens, capacity):
    i = pl.program_id(0)
    e = n_experts

    hist = hist_ref[...]
    counts = jnp.zeros((e, 1), jnp.float32)
    kmax = jnp.zeros((e, 1), jnp.float32)
    for k in range(_K):
        a = (counts < capacity).astype(jnp.float32)
        kmax = kmax + a
        counts = counts + hist[:, k:k + 1] * a

    iota_e = jax.lax.broadcasted_iota(jnp.int32, (e, 1), 0)
    minc = jnp.min(counts, axis=0, keepdims=True)
    least = jnp.min(jnp.where(counts == minc, iota_e, e), axis=0,
                    keepdims=True)

    d0 = d0_ref[...]
    kmat = kmat_ref[...]
    dm = d0 * (kmat < kmax).astype(jnp.float32)
    rs = jnp.sum(dm, axis=0, keepdims=True)
    iota_s = jax.lax.broadcasted_iota(jnp.int32, dm.shape, 0)
    dm = jnp.where((rs == 0.0) & (iota_s == least), 1.0, dm)
    dm = dm / (jnp.sum(dm, axis=0, keepdims=True) + 1e-8)
    dm_ref[...] = jnp.swapaxes(dm, 0, 1)

    @pl.when(i == 0)
    def _init():
        colsum_ref[...] = jnp.zeros_like(colsum_ref)

    colsum_ref[...] += jnp.sum(dm, axis=1, keepdims=True)

    @pl.when(i == pl.num_programs(0) - 1)
    def _final():
        counts2 = colsum_ref[...]
        target = n_tokens * _K / e
        lb = jnp.mean(jnp.square(counts2 / n_tokens - target / n_tokens))
        z = zsum_ref[0, 0] / (n_tokens * e)
        loss_ref[...] = jnp.full((1, 1), 0.0, jnp.float32) + (
            0.001 * z + 0.001 * lb)


@jax.jit
def kernel(x, W):
    b, s, d = x.shape
    n = b * s
    e = W.shape[0]
    capacity = int(_CAPACITY_FACTOR * n * _K / e)
    xf = x.reshape(n, d)

    bn_a = 1024
    grid_a = n // bn_a
    rw, d0, kmat, hist, zsum = pl.pallas_call(
        functools.partial(_phase_a_body, n_experts=e),
        grid=(grid_a,),
        in_specs=[
            pl.BlockSpec((bn_a, d), lambda i: (i, 0)),
            pl.BlockSpec((e, d), lambda i: (0, 0)),
        ],
        out_specs=[
            pl.BlockSpec((bn_a, e), lambda i: (i, 0)),
            pl.BlockSpec((e, bn_a), lambda i: (0, i)),
            pl.BlockSpec((e, bn_a), lambda i: (0, i)),
            pl.BlockSpec((e, _K), lambda i: (0, 0)),
            pl.BlockSpec((1, 1), lambda i: (0, 0)),
        ],
        out_shape=[
            jax.ShapeDtypeStruct((n, e), jnp.float32),
            jax.ShapeDtypeStruct((e, n), jnp.float32),
            jax.ShapeDtypeStruct((e, n), jnp.float32),
            jax.ShapeDtypeStruct((e, _K), jnp.float32),
            jax.ShapeDtypeStruct((1, 1), jnp.float32),
        ],
        input_output_aliases={},
    )(xf, W)

    bn_b = 2048
    grid_b = n // bn_b
    dm, loss = pl.pallas_call(
        functools.partial(_phase_b_body, n_experts=e, n_tokens=n,
                          capacity=capacity),
        grid=(grid_b,),
        in_specs=[
            pl.BlockSpec((e, bn_b), lambda i: (0, i)),
            pl.BlockSpec((e, bn_b), lambda i: (0, i)),
            pl.BlockSpec((e, _K), lambda i: (0, 0)),
            pl.BlockSpec((1, 1), lambda i: (0, 0)),
        ],
        out_specs=[
            pl.BlockSpec((bn_b, e), lambda i: (i, 0)),
            pl.BlockSpec((1, 1), lambda i: (0, 0)),
        ],
        out_shape=[
            jax.ShapeDtypeStruct((n, e), jnp.float32),
            jax.ShapeDtypeStruct((1, 1), jnp.float32),
        ],
        scratch_shapes=[pltpu.VMEM((e, 1), jnp.float32)],
    )(d0, kmat, hist, zsum)

    return rw, dm, loss[0, 0]


# speedup vs baseline: 17.1040x; 1.1402x over previous
import functools

import jax
import jax.numpy as jnp
from jax.experimental import pallas as pl
from jax.experimental.pallas import tpu as pltpu

_K = 8
_CAPACITY_FACTOR = 1.25


def _fused_body(x_ref, w_ref, rw_ref, dm_ref, loss_ref,
                d0_s, kmat_s, hist_s, zsum_s,
                *, n_experts, n_tokens, capacity, bn_a, grid_a):
    i = pl.program_id(0)
    e = n_experts

    @pl.when(i == 0)
    def _init():
        hist_s[...] = jnp.zeros_like(hist_s)
        zsum_s[...] = jnp.zeros_like(zsum_s)

    @pl.when(i < grid_a)
    def _phase_a():
        logits = jax.lax.dot_general(
            w_ref[...], x_ref[...], (((1,), (1,)), ((), ())),
            preferred_element_type=jnp.float32)
        zsum_s[...] += jnp.sum(logits * logits)

        m = jnp.max(logits, axis=0, keepdims=True)
        ex = jnp.exp(logits - m)
        p = ex / jnp.sum(ex, axis=0, keepdims=True)
        rw_ref[...] = jnp.swapaxes(p, 0, 1)

        iota = jax.lax.broadcasted_iota(jnp.int32, p.shape, 0)
        work = p
        kacc = jnp.full(p.shape, float(_K), jnp.float32)
        wsum = jnp.zeros((1, p.shape[1]), jnp.float32)
        hcols = []
        for k in range(_K):
            mk = jnp.max(work, axis=0, keepdims=True)
            idx = jnp.min(jnp.where(work == mk, iota, e), axis=0,
                          keepdims=True)
            oh = (iota == idx).astype(jnp.float32)
            wsum = wsum + mk
            kacc = kacc - (float(_K) - k) * oh
            hcols.append(jnp.sum(oh, axis=1, keepdims=True))
            work = work - oh * (work + 1.0)
        kmat_s[:, pl.ds(i * bn_a, bn_a)] = kacc
        d0_s[:, pl.ds(i * bn_a, bn_a)] = p / (wsum + 1e-8)
        hist_s[...] += jnp.concatenate(hcols, axis=1)

    @pl.when(i == grid_a)
    def _phase_b():
        hist = hist_s[...]
        counts = jnp.zeros((e, 1), jnp.float32)
        kmax = jnp.zeros((e, 1), jnp.float32)
        for k in range(_K):
            a = (counts < capacity).astype(jnp.float32)
            kmax = kmax + a
            counts = counts + hist[:, k:k + 1] * a

        iota_e = jax.lax.broadcasted_iota(jnp.int32, (e, 1), 0)
        minc = jnp.min(counts, axis=0, keepdims=True)
        least = jnp.min(jnp.where(counts == minc, iota_e, e), axis=0,
                        keepdims=True)

        d0 = d0_s[...]
        kmat = kmat_s[...]
        dm = d0 * (kmat < kmax).astype(jnp.float32)
        rs = jnp.sum(dm, axis=0, keepdims=True)
        iota_s = jax.lax.broadcasted_iota(jnp.int32, dm.shape, 0)
        dm = jnp.where((rs == 0.0) & (iota_s == least), 1.0, dm)
        dm = dm / (jnp.sum(dm, axis=0, keepdims=True) + 1e-8)
        dm_ref[...] = jnp.swapaxes(dm, 0, 1)

        counts2 = jnp.sum(dm, axis=1, keepdims=True)
        target = n_tokens * _K / e
        lb = jnp.mean(jnp.square(counts2 / n_tokens - target / n_tokens))
        z = zsum_s[0, 0] / (n_tokens * e)
        loss_ref[...] = jnp.full((1, 1), 0.0, jnp.float32) + (
            0.001 * z + 0.001 * lb)


@jax.jit
def kernel(x, W):
    b, s, d = x.shape
    n = b * s
    e = W.shape[0]
    capacity = int(_CAPACITY_FACTOR * n * _K / e)
    xf = x.reshape(n, d)

    bn_a = 1024
    grid_a = n // bn_a
    last = grid_a - 1
    rw, dm, loss = pl.pallas_call(
        functools.partial(_fused_body, n_experts=e, n_tokens=n,
                          capacity=capacity, bn_a=bn_a, grid_a=grid_a),
        grid=(grid_a + 1,),
        in_specs=[
            pl.BlockSpec((bn_a, d), lambda i: (jnp.minimum(i, last), 0)),
            pl.BlockSpec((e, d), lambda i: (0, 0)),
        ],
        out_specs=[
            pl.BlockSpec((bn_a, e), lambda i: (jnp.minimum(i, last), 0)),
            pl.BlockSpec((n, e), lambda i: (0, 0)),
            pl.BlockSpec((1, 1), lambda i: (0, 0)),
        ],
        out_shape=[
            jax.ShapeDtypeStruct((n, e), jnp.float32),
            jax.ShapeDtypeStruct((n, e), jnp.float32),
            jax.ShapeDtypeStruct((1, 1), jnp.float32),
        ],
        scratch_shapes=[
            pltpu.VMEM((e, n), jnp.float32),
            pltpu.VMEM((e, n), jnp.float32),
            pltpu.VMEM((e, _K), jnp.float32),
            pltpu.VMEM((1, 1), jnp.float32),
        ],
    )(xf, W)

    return rw, dm, loss[0, 0]
